# Initial kernel scaffold; baseline (speedup 1.0000x reference)
import jax
import jax.numpy as jnp
from jax.experimental import pallas as pl


def kernel(x, pos, edge_index, batch, params):
    raise NotImplementedError("write your pallas kernel here")


# speedup vs baseline: 3.2987x; 3.2987x over previous
import jax
import jax.numpy as jnp
from jax import lax
from jax.experimental import pallas as pl
from jax.experimental.pallas import tpu as pltpu
from jax.experimental.pallas import tpu_sc as plsc

N = 50000
NP = 50048
E = 800000
H = 64
HH = 32
EB = E // 128
NS = 16
ROWS_PER_SUB = NP // NS
ZROWS = 136
BS = 3128
NB = NP // BS
EBS = 3200
NEB = E // EBS

_MESH = plsc.VectorSubcoreMesh(core_axis_name="c", subcore_axis_name="s")
_CP = pltpu.CompilerParams(use_tc_tiling_on_sc=False)


def _zero_shared(z_ref, sp_ref, width, rows0, zrows, niter):
    @pl.loop(0, zrows)
    def _(r):
        for j in range(0, width, 16):
            z_ref.at[pl.ds(r, 1), pl.ds(j, 16)][...] = jnp.zeros(
                (1, 16), jnp.float32)

    @pl.loop(0, niter)
    def _(i):
        pltpu.sync_copy(z_ref, sp_ref.at[pl.ds(rows0 + i * zrows, zrows)])


def _edge_kernel(a2, b2, cf, dsth, srch, r_out,
                 R_sp, dbuf, sbuf, aidx, sidx, abuf, bbuf, cbuf, rbuf,
                 zbuf, sem_a, sem_b, sem_c):
    c = lax.axis_index("c")
    s = lax.axis_index("s")
    rows0 = s * ROWS_PER_SUB
    _zero_shared(zbuf, R_sp, HH, rows0, ZROWS, ROWS_PER_SUB // ZROWS)
    plsc.subcore_barrier()

    cNP = c * NP
    nblk = jnp.where(s < EB - NS * (EB // NS), EB // NS + 1, EB // NS)
    blk0 = s * (EB // NS) + jnp.minimum(s, EB - NS * (EB // NS))

    @pl.loop(0, nblk)
    def _(i):
        blk = blk0 + i
        pltpu.sync_copy(dsth.at[blk], dbuf)
        pltpu.sync_copy(srch.at[blk], sbuf)
        for j in range(0, 128, 16):
            sl = (pl.ds(0, 1), pl.ds(j, 16))
            aidx.at[sl][...] = dbuf.at[sl][...] + cNP
            sidx.at[sl][...] = sbuf.at[sl][...] + cNP
        ca = pltpu.async_copy(a2.at[aidx.at[0]], abuf, sem_a)
        cb = pltpu.async_copy(b2.at[sidx.at[0]], bbuf, sem_b)
        cc = pltpu.async_copy(cf.at[c * EB + blk], cbuf, sem_c)
        ca.wait()
        cb.wait()
        cc.wait()

        @pl.loop(0, 128)
        def _(r):
            for j in range(0, HH, 16):
                sl = (pl.ds(r, 1), pl.ds(j, 16))
                v = abuf.at[sl][...] + bbuf.at[sl][...] + cbuf.at[sl][...]
                rbuf.at[sl][...] = jnp.maximum(v, 0.0)

        pltpu.sync_copy(rbuf, R_sp.at[dbuf.at[0]], add=True)

    plsc.subcore_barrier()
    pltpu.sync_copy(R_sp.at[pl.ds(rows0, ROWS_PER_SUB)],
                    r_out.at[pl.ds(c * NP + rows0, ROWS_PER_SUB)])


def _sc_edge_pass(a2, b2, cf, dsth, srch):
    k = pl.kernel(
        _edge_kernel,
        out_type=jax.ShapeDtypeStruct((2 * NP, HH), jnp.float32),
        mesh=_MESH,
        scratch_types=[
            pltpu.VMEM_SHARED((NP, HH), jnp.float32),
            pltpu.VMEM((1, 128), jnp.int32),
            pltpu.VMEM((1, 128), jnp.int32),
            pltpu.VMEM((1, 128), jnp.int32),
            pltpu.VMEM((1, 128), jnp.int32),
            pltpu.VMEM((128, HH), jnp.float32),
            pltpu.VMEM((128, HH), jnp.float32),
            pltpu.VMEM((128, HH), jnp.float32),
            pltpu.VMEM((128, HH), jnp.float32),
            pltpu.VMEM((ZROWS, HH), jnp.float32),
            pltpu.SemaphoreType.DMA,
            pltpu.SemaphoreType.DMA,
            pltpu.SemaphoreType.DMA,
        ],
        compiler_params=_CP,
    )
    return k(a2, b2, cf, dsth, srch)


def _posdeg_kernel(posp, dsth, srch, p_out, deg_out,
                   D_sp, dbuf, sbuf, pibuf, pjbuf, onesb, zbuf,
                   sem_i, sem_j):
    c = lax.axis_index("c")
    s = lax.axis_index("s")
    rows0 = s * ROWS_PER_SUB
    _zero_shared(zbuf, D_sp, 16, rows0, ZROWS, ROWS_PER_SUB // ZROWS)

    @pl.loop(0, 128)
    def _(r):
        onesb.at[pl.ds(r, 1), pl.ds(0, 16)][...] = jnp.ones((1, 16),
                                                            jnp.float32)
    plsc.subcore_barrier()

    w = c * NS + s
    nw = 2 * NS
    nblk = jnp.where(w < EB - nw * (EB // nw), EB // nw + 1, EB // nw)
    blk0 = w * (EB // nw) + jnp.minimum(w, EB - nw * (EB // nw))

    @pl.loop(0, nblk)
    def _(i):
        blk = blk0 + i
        pltpu.sync_copy(dsth.at[blk], dbuf)
        pltpu.sync_copy(srch.at[blk], sbuf)
        ci = pltpu.async_copy(posp.at[dbuf.at[0]], pibuf, sem_i)
        cj = pltpu.async_copy(posp.at[sbuf.at[0]], pjbuf, sem_j)
        ci.wait()
        cj.wait()
        pltpu.sync_copy(pibuf, p_out.at[pl.ds(blk * 128, 128)])
        pltpu.sync_copy(pjbuf, p_out.at[pl.ds(E + blk * 128, 128)])
        pltpu.sync_copy(onesb, D_sp.at[dbuf.at[0]], add=True)

    plsc.subcore_barrier()
    pltpu.sync_copy(D_sp.at[pl.ds(rows0, ROWS_PER_SUB)],
                    deg_out.at[pl.ds(c * NP + rows0, ROWS_PER_SUB)])


def _sc_posdeg(posp, dsth, srch):
    k = pl.kernel(
        _posdeg_kernel,
        out_type=[
            jax.ShapeDtypeStruct((2 * E, 16), jnp.float32),
            jax.ShapeDtypeStruct((2 * NP, 16), jnp.float32),
        ],
        mesh=_MESH,
        scratch_types=[
            pltpu.VMEM_SHARED((NP, 16), jnp.float32),
            pltpu.VMEM((1, 128), jnp.int32),
            pltpu.VMEM((1, 128), jnp.int32),
            pltpu.VMEM((128, 16), jnp.float32),
            pltpu.VMEM((128, 16), jnp.float32),
            pltpu.VMEM((128, 16), jnp.float32),
            pltpu.VMEM((ZROWS, 16), jnp.float32),
            pltpu.SemaphoreType.DMA,
            pltpu.SemaphoreType.DMA,
        ],
        compiler_params=_CP,
    )
    return k(posp, dsth, srch)


def _p0_body(x_ref, wi_ref, bi_ref, h_ref):
    h_ref[...] = jax.nn.relu(x_ref[...] @ wi_ref[...] + bi_ref[...])


def _p0(xp, wi, bi):
    return pl.pallas_call(
        _p0_body,
        grid=(NB,),
        in_specs=[
            pl.BlockSpec((BS, 8), lambda i: (i, 0)),
            pl.BlockSpec((8, H), lambda i: (0, 0)),
            pl.BlockSpec((1, H), lambda i: (0, 0)),
        ],
        out_specs=pl.BlockSpec((BS, H), lambda i: (i, 0)),
        out_shape=jax.ShapeDtypeStruct((NP, H), jnp.float32),
    )(xp, wi, bi)


def _p1_body(h_ref, wa_ref, wb_ref, a_ref, b_ref):
    hb = h_ref[...]
    a_ref[...] = hb @ wa_ref[0]
    b_ref[...] = hb @ wb_ref[0]


def _p1(h, wa, wb):
    return pl.pallas_call(
        _p1_body,
        grid=(2, NB),
        in_specs=[
            pl.BlockSpec((BS, H), lambda c, i: (i, 0)),
            pl.BlockSpec((1, H, HH), lambda c, i: (c, 0, 0)),
            pl.BlockSpec((1, H, HH), lambda c, i: (c, 0, 0)),
        ],
        out_specs=[
            pl.BlockSpec((BS, HH), lambda c, i: (c * NB + i, 0)),
            pl.BlockSpec((BS, HH), lambda c, i: (c * NB + i, 0)),
        ],
        out_shape=[
            jax.ShapeDtypeStruct((2 * NP, HH), jnp.float32),
            jax.ShapeDtypeStruct((2 * NP, HH), jnp.float32),
        ],
    )(h, wa, wb)


def _pc_body(pi_ref, pj_ref, wc_ref, b1_ref, c_ref):
    pi = pi_ref[...]
    pj = pj_ref[...]
    diff = pi[:, :3] - pj[:, :3]
    d2 = jnp.sum(diff * diff, axis=1, keepdims=True)
    dist = jnp.sqrt(d2)
    unit = diff / (dist + 1e-8)
    geo = jnp.concatenate(
        [dist, unit, dist * dist, d2, jnp.zeros((EBS, 2), jnp.float32)],
        axis=1)
    c_ref[...] = (geo @ wc_ref[0] + b1_ref[0]).reshape(
        EBS // 128, 128, HH)


def _pc(p_all, wc, b1):
    return pl.pallas_call(
        _pc_body,
        grid=(2, NEB),
        in_specs=[
            pl.BlockSpec((EBS, 16), lambda c, i: (i, 0)),
            pl.BlockSpec((EBS, 16), lambda c, i: (NEB + i, 0)),
            pl.BlockSpec((1, 8, HH), lambda c, i: (c, 0, 0)),
            pl.BlockSpec((1, 1, HH), lambda c, i: (c, 0, 0)),
        ],
        out_specs=pl.BlockSpec((EBS // 128, 128, HH),
                               lambda c, i: (c * NEB + i, 0, 0)),
        out_shape=jax.ShapeDtypeStruct((2 * EB, 128, HH), jnp.float32),
    )(p_all, p_all, wc, b1)


def _p2_body(rlo_ref, rhi_ref, h_ref, dlo_ref, dhi_ref, w2_ref, b2_ref,
             u1a_ref, u1b_ref, bu1_ref, u2_ref, bu2_ref, g_ref, be_ref,
             o_ref):
    R = jnp.concatenate([rlo_ref[...], rhi_ref[...]], axis=1)
    deg = dlo_ref[:, :1] + dhi_ref[:, :1]
    aggr = R @ w2_ref[...] + deg * b2_ref[...]
    h = h_ref[...]
    u_pre = h @ u1a_ref[...] + aggr @ u1b_ref[...] + bu1_ref[...]
    out = jax.nn.relu(u_pre) @ u2_ref[...] + bu2_ref[...]
    mu = jnp.mean(out, axis=-1, keepdims=True)
    var = jnp.mean((out - mu) ** 2, axis=-1, keepdims=True)
    out = (out - mu) / jnp.sqrt(var + 1e-5) * g_ref[...] + be_ref[...]
    o_ref[...] = jax.nn.relu(out) + h


def _p2(r2, h, deg2, w2, b2, u1a, u1b, bu1, u2, bu2, g, be):
    mm = lambda i: (0, 0)
    return pl.pallas_call(
        _p2_body,
        grid=(NB,),
        in_specs=[
            pl.BlockSpec((BS, HH), lambda i: (i, 0)),
            pl.BlockSpec((BS, HH), lambda i: (NB + i, 0)),
            pl.BlockSpec((BS, H), lambda i: (i, 0)),
            pl.BlockSpec((BS, 16), lambda i: (i, 0)),
            pl.BlockSpec((BS, 16), lambda i: (NB + i, 0)),
            pl.BlockSpec((H, H), mm),
            pl.BlockSpec((1, H), mm),
            pl.BlockSpec((H, H), mm),
            pl.BlockSpec((H, H), mm),
            pl.BlockSpec((1, H), mm),
            pl.BlockSpec((H, H), mm),
            pl.BlockSpec((1, H), mm),
            pl.BlockSpec((1, H), mm),
            pl.BlockSpec((1, H), mm),
        ],
        out_specs=pl.BlockSpec((BS, H), lambda i: (i, 0)),
        out_shape=jax.ShapeDtypeStruct((NP, H), jnp.float32),
    )(r2, r2, h, deg2, deg2, w2, b2, u1a, u1b, bu1, u2, bu2, g, be)


def _p3_body(h_ref, bt_ref, wo_ref, bo_ref, g1_ref, bg1_ref, g2_ref,
             bg2_ref, o_ref, acc, cnt):
    i = pl.program_id(0)

    @pl.when(i == 0)
    def _():
        acc[...] = jnp.zeros_like(acc)
        cnt[...] = jnp.zeros_like(cnt)

    xo = h_ref[...] @ wo_ref[...] + bo_ref[...]
    bt = bt_ref[...]
    gid = lax.broadcasted_iota(jnp.int32, (BS, 8), 1)
    rm = (bt == gid).astype(jnp.float32)
    dn = (((0,), (0,)), ((), ()))
    acc[...] += lax.dot_general(rm, xo, dn)
    cnt[...] += lax.dot_general(rm, jnp.ones_like(xo), dn)

    @pl.when(i == NB - 1)
    def _():
        cts = cnt[...]
        means = jnp.where(cts > 0, acc[...] / jnp.maximum(cts, 1.0), 0.0)
        gm = jax.nn.relu(means @ g1_ref[...] + bg1_ref[...])
        o_ref[...] = gm @ g2_ref[...] + bg2_ref[...]


def _p3(h, btp, wo, bo, g1, bg1, g2, bg2):
    mm = lambda i: (0, 0)
    return pl.pallas_call(
        _p3_body,
        grid=(NB,),
        in_specs=[
            pl.BlockSpec((BS, H), lambda i: (i, 0)),
            pl.BlockSpec((BS, 1), lambda i: (i, 0)),
            pl.BlockSpec((H, 128), mm),
            pl.BlockSpec((1, 128), mm),
            pl.BlockSpec((128, H), mm),
            pl.BlockSpec((1, H), mm),
            pl.BlockSpec((H, 128), mm),
            pl.BlockSpec((1, 128), mm),
        ],
        out_specs=pl.BlockSpec((8, 128), mm),
        out_shape=jax.ShapeDtypeStruct((8, 128), jnp.float32),
        scratch_shapes=[
            pltpu.VMEM((8, 128), jnp.float32),
            pltpu.VMEM((8, 128), jnp.float32),
        ],
    )(h, btp, wo, bo, g1, bg1, g2, bg2)


def kernel(x, pos, edge_index, batch, params):
    dsth = edge_index[1].astype(jnp.int32).reshape(EB, 1, 128)
    srch = edge_index[0].astype(jnp.int32).reshape(EB, 1, 128)
    posp = jnp.concatenate([pos, jnp.zeros((N, 13), jnp.float32)], axis=1)
    p_all, deg2 = _sc_posdeg(posp, dsth, srch)

    xp = jnp.concatenate([
        jnp.concatenate([x, jnp.zeros((N, 5), jnp.float32)], axis=1),
        jnp.zeros((NP - N, 8), jnp.float32)], axis=0)
    wi = jnp.concatenate([params["inp"]["w"],
                          jnp.zeros((5, H), jnp.float32)], axis=0)
    h = _p0(xp, wi, params["inp"]["b"][None])

    btp = jnp.concatenate([batch.astype(jnp.int32),
                           jnp.full((NP - N,), 8, jnp.int32)])[:, None]

    for lp in params["layers"]:
        W1 = lp["m1"]["w"]
        wc6 = jnp.concatenate([W1[2 * H:], jnp.zeros((2, H), jnp.float32)],
                              axis=0)
        wc = jnp.stack([wc6[:, :HH], wc6[:, HH:]], axis=0)
        b1 = lp["m1"]["b"][None]
        b1s = jnp.stack([b1[:, :HH], b1[:, HH:]], axis=0)
        wa = jnp.stack([W1[:H, :HH], W1[:H, HH:]], axis=0)
        wb = jnp.stack([W1[H:2 * H, :HH], W1[H:2 * H, HH:]], axis=0)
        a2, b2t = _p1(h, wa, wb)
        cf = _pc(p_all, wc, b1s)
        r2 = _sc_edge_pass(a2, b2t, cf, dsth, srch)
        h = _p2(r2, h, deg2, lp["m2"]["w"], lp["m2"]["b"][None],
                lp["u1"]["w"][:H], lp["u1"]["w"][H:], lp["u1"]["b"][None],
                lp["u2"]["w"], lp["u2"]["b"][None],
                lp["ln_g"][None], lp["ln_b"][None])

    return _p3(h, btp, params["outp"]["w"], params["outp"]["b"][None],
               params["gp1"]["w"], params["gp1"]["b"][None],
               params["gp2"]["w"], params["gp2"]["b"][None])


# speedup vs baseline: 3.4630x; 1.0498x over previous
import jax
import jax.numpy as jnp
from jax import lax
from jax.experimental import pallas as pl
from jax.experimental.pallas import tpu as pltpu
from jax.experimental.pallas import tpu_sc as plsc

N = 50000
NP = 50048
E = 800000
H = 64
HH = 32
EB = E // 128
NS = 16
ROWS_PER_SUB = NP // NS
ZROWS = 136
BS = 3128
NB = NP // BS
EBS = 3200
NEB = E // EBS

_MESH = plsc.VectorSubcoreMesh(core_axis_name="c", subcore_axis_name="s")
_CP = pltpu.CompilerParams(use_tc_tiling_on_sc=False)


def _zero_shared(z_ref, sp_ref, width, rows0, zrows, niter):
    @pl.loop(0, zrows)
    def _(r):
        for j in range(0, width, 16):
            z_ref.at[pl.ds(r, 1), pl.ds(j, 16)][...] = jnp.zeros(
                (1, 16), jnp.float32)

    @pl.loop(0, niter)
    def _(i):
        pltpu.sync_copy(z_ref, sp_ref.at[pl.ds(rows0 + i * zrows, zrows)])


SUP = 4


def _edge_kernel(a2, b2, cf, dsth, srch, zrows_hbm, r_out,
                 R_sp, dbuf16, sbuf16, aidx16, sidx16,
                 ab0, ab1, bb0, bb1, cb0, cb1,
                 sem_a0, sem_a1, sem_b0, sem_b1, sem_c0, sem_c1,
                 sem_s0, sem_s1):
    c = lax.axis_index("c")
    s = lax.axis_index("s")
    rows0 = s * ROWS_PER_SUB
    pltpu.sync_copy(zrows_hbm.at[pl.ds(rows0, ROWS_PER_SUB)],
                    R_sp.at[pl.ds(rows0, ROWS_PER_SUB)])
    plsc.subcore_barrier()

    cNP = c * NP
    nblk = jnp.where(s < EB - NS * (EB // NS), EB // NS + 1, EB // NS)
    blk0 = s * (EB // NS) + jnp.minimum(s, EB - NS * (EB // NS))
    nsupf = nblk // SUP

    abufs = (ab0, ab1)
    bbufs = (bb0, bb1)
    cbufs = (cb0, cb1)
    sems_a = (sem_a0, sem_a1)
    sems_b = (sem_b0, sem_b1)
    sems_c = (sem_c0, sem_c1)
    sems_s = (sem_s0, sem_s1)

    def compute_block(ab, bb, cb):
        @pl.loop(0, 128)
        def _(r):
            for j in range(0, HH, 16):
                sl = (pl.ds(r, 1), pl.ds(j, 16))
                v = ab.at[sl][...] + bb.at[sl][...] + cb.at[sl][...]
                ab.at[sl][...] = jnp.maximum(v, 0.0)

    @pl.loop(0, nsupf)
    def _(sp):
        base = blk0 + sp * SUP
        pltpu.sync_copy(dsth.at[pl.ds(base, SUP)], dbuf16)
        pltpu.sync_copy(srch.at[pl.ds(base, SUP)], sbuf16)
        for k in range(SUP):
            for j in range(0, 128, 16):
                sl = (pl.ds(k, 1), pl.ds(j, 16))
                aidx16.at[sl][...] = dbuf16.at[sl][...] + cNP
                sidx16.at[sl][...] = sbuf16.at[sl][...] + cNP

        def fire(k):
            t = k % 2
            return (
                pltpu.async_copy(a2.at[aidx16.at[k]], abufs[t], sems_a[t]),
                pltpu.async_copy(b2.at[sidx16.at[k]], bbufs[t], sems_b[t]),
                pltpu.async_copy(cf.at[c * EB + base + k], cbufs[t],
                                 sems_c[t]),
            )

        gh = fire(0)
        sc_h = [None, None]
        for k in range(SUP):
            t = k % 2
            t2 = (k + 1) % 2
            nxt = None
            if k + 1 < SUP:
                if sc_h[t2] is not None:
                    sc_h[t2].wait()
                    sc_h[t2] = None
                nxt = fire(k + 1)
            for hnd in gh:
                hnd.wait()
            compute_block(abufs[t], bbufs[t], cbufs[t])
            sc_h[t] = pltpu.async_copy(abufs[t], R_sp.at[dbuf16.at[k]],
                                       sems_s[t], add=True)
            gh = nxt
        for t in (0, 1):
            if sc_h[t] is not None:
                sc_h[t].wait()

    @pl.loop(0, nblk - nsupf * SUP)
    def _(i):
        blk = blk0 + nsupf * SUP + i
        pltpu.sync_copy(dsth.at[pl.ds(blk, 1)], dbuf16.at[pl.ds(0, 1)])
        pltpu.sync_copy(srch.at[pl.ds(blk, 1)], sbuf16.at[pl.ds(0, 1)])
        for j in range(0, 128, 16):
            sl = (pl.ds(0, 1), pl.ds(j, 16))
            aidx16.at[sl][...] = dbuf16.at[sl][...] + cNP
            sidx16.at[sl][...] = sbuf16.at[sl][...] + cNP
        ca = pltpu.async_copy(a2.at[aidx16.at[0]], ab0, sem_a0)
        cb = pltpu.async_copy(b2.at[sidx16.at[0]], bb0, sem_b0)
        cc = pltpu.async_copy(cf.at[c * EB + blk], cb0, sem_c0)
        ca.wait()
        cb.wait()
        cc.wait()
        compute_block(ab0, bb0, cb0)
        pltpu.sync_copy(ab0, R_sp.at[dbuf16.at[0]], add=True)

    plsc.subcore_barrier()
    pltpu.sync_copy(R_sp.at[pl.ds(rows0, ROWS_PER_SUB)],
                    r_out.at[pl.ds(c * NP + rows0, ROWS_PER_SUB)])


def _sc_edge_pass(a2, b2, cf, dsth, srch, zrows):
    k = pl.kernel(
        _edge_kernel,
        out_type=jax.ShapeDtypeStruct((2 * NP, HH), jnp.float32),
        mesh=_MESH,
        scratch_types=[
            pltpu.VMEM_SHARED((NP, HH), jnp.float32),
            pltpu.VMEM((SUP, 128), jnp.int32),
            pltpu.VMEM((SUP, 128), jnp.int32),
            pltpu.VMEM((SUP, 128), jnp.int32),
            pltpu.VMEM((SUP, 128), jnp.int32),
            pltpu.VMEM((128, HH), jnp.float32),
            pltpu.VMEM((128, HH), jnp.float32),
            pltpu.VMEM((128, HH), jnp.float32),
            pltpu.VMEM((128, HH), jnp.float32),
            pltpu.VMEM((128, HH), jnp.float32),
            pltpu.VMEM((128, HH), jnp.float32),
            pltpu.SemaphoreType.DMA,
            pltpu.SemaphoreType.DMA,
            pltpu.SemaphoreType.DMA,
            pltpu.SemaphoreType.DMA,
            pltpu.SemaphoreType.DMA,
            pltpu.SemaphoreType.DMA,
            pltpu.SemaphoreType.DMA,
            pltpu.SemaphoreType.DMA,
        ],
        compiler_params=_CP,
    )
    return k(a2, b2, cf, dsth, srch, zrows)


def _posdeg_kernel(posp, dsth, srch, p_out, deg_out,
                   D_sp, dbuf, sbuf, pibuf, pjbuf, onesb, zbuf,
                   sem_i, sem_j):
    c = lax.axis_index("c")
    s = lax.axis_index("s")
    rows0 = s * ROWS_PER_SUB
    _zero_shared(zbuf, D_sp, 16, rows0, ZROWS, ROWS_PER_SUB // ZROWS)

    @pl.loop(0, 128)
    def _(r):
        onesb.at[pl.ds(r, 1), pl.ds(0, 16)][...] = jnp.ones((1, 16),
                                                            jnp.float32)
    plsc.subcore_barrier()

    w = c * NS + s
    nw = 2 * NS
    nblk = jnp.where(w < EB - nw * (EB // nw), EB // nw + 1, EB // nw)
    blk0 = w * (EB // nw) + jnp.minimum(w, EB - nw * (EB // nw))

    @pl.loop(0, nblk)
    def _(i):
        blk = blk0 + i
        pltpu.sync_copy(dsth.at[pl.ds(blk, 1)], dbuf)
        pltpu.sync_copy(srch.at[pl.ds(blk, 1)], sbuf)
        ci = pltpu.async_copy(posp.at[dbuf.at[0]], pibuf, sem_i)
        cj = pltpu.async_copy(posp.at[sbuf.at[0]], pjbuf, sem_j)
        ci.wait()
        cj.wait()
        pltpu.sync_copy(pibuf, p_out.at[pl.ds(blk * 128, 128)])
        pltpu.sync_copy(pjbuf, p_out.at[pl.ds(E + blk * 128, 128)])
        pltpu.sync_copy(onesb, D_sp.at[dbuf.at[0]], add=True)

    plsc.subcore_barrier()
    pltpu.sync_copy(D_sp.at[pl.ds(rows0, ROWS_PER_SUB)],
                    deg_out.at[pl.ds(c * NP + rows0, ROWS_PER_SUB)])


def _sc_posdeg(posp, dsth, srch):
    k = pl.kernel(
        _posdeg_kernel,
        out_type=[
            jax.ShapeDtypeStruct((2 * E, 16), jnp.float32),
            jax.ShapeDtypeStruct((2 * NP, 16), jnp.float32),
        ],
        mesh=_MESH,
        scratch_types=[
            pltpu.VMEM_SHARED((NP, 16), jnp.float32),
            pltpu.VMEM((1, 128), jnp.int32),
            pltpu.VMEM((1, 128), jnp.int32),
            pltpu.VMEM((128, 16), jnp.float32),
            pltpu.VMEM((128, 16), jnp.float32),
            pltpu.VMEM((128, 16), jnp.float32),
            pltpu.VMEM((ZROWS, 16), jnp.float32),
            pltpu.SemaphoreType.DMA,
            pltpu.SemaphoreType.DMA,
        ],
        compiler_params=_CP,
    )
    return k(posp, dsth, srch)


def _p0_body(x_ref, wi_ref, bi_ref, h_ref):
    h_ref[...] = jax.nn.relu(x_ref[...] @ wi_ref[...] + bi_ref[...])


def _p0(xp, wi, bi):
    return pl.pallas_call(
        _p0_body,
        grid=(NB,),
        in_specs=[
            pl.BlockSpec((BS, 8), lambda i: (i, 0)),
            pl.BlockSpec((8, H), lambda i: (0, 0)),
            pl.BlockSpec((1, H), lambda i: (0, 0)),
        ],
        out_specs=pl.BlockSpec((BS, H), lambda i: (i, 0)),
        out_shape=jax.ShapeDtypeStruct((NP, H), jnp.float32),
    )(xp, wi, bi)


def _p1_body(h_ref, wa_ref, wb_ref, a_ref, b_ref):
    hb = h_ref[...]
    a_ref[...] = hb @ wa_ref[0]
    b_ref[...] = hb @ wb_ref[0]


def _p1(h, wa, wb):
    return pl.pallas_call(
        _p1_body,
        grid=(2, NB),
        in_specs=[
            pl.BlockSpec((BS, H), lambda c, i: (i, 0)),
            pl.BlockSpec((1, H, HH), lambda c, i: (c, 0, 0)),
            pl.BlockSpec((1, H, HH), lambda c, i: (c, 0, 0)),
        ],
        out_specs=[
            pl.BlockSpec((BS, HH), lambda c, i: (c * NB + i, 0)),
            pl.BlockSpec((BS, HH), lambda c, i: (c * NB + i, 0)),
        ],
        out_shape=[
            jax.ShapeDtypeStruct((2 * NP, HH), jnp.float32),
            jax.ShapeDtypeStruct((2 * NP, HH), jnp.float32),
        ],
    )(h, wa, wb)


def _pc_body(pi_ref, pj_ref, wc_ref, b1_ref, c_ref):
    pi = pi_ref[...]
    pj = pj_ref[...]
    diff = pi[:, :3] - pj[:, :3]
    d2 = jnp.sum(diff * diff, axis=1, keepdims=True)
    dist = jnp.sqrt(d2)
    unit = diff / (dist + 1e-8)
    geo = jnp.concatenate(
        [dist, unit, dist * dist, d2, jnp.zeros((EBS, 2), jnp.float32)],
        axis=1)
    c_ref[...] = (geo @ wc_ref[0] + b1_ref[0]).reshape(
        EBS // 128, 128, HH)


def _pc(p_all, wc, b1):
    return pl.pallas_call(
        _pc_body,
        grid=(2, NEB),
        in_specs=[
            pl.BlockSpec((EBS, 16), lambda c, i: (i, 0)),
            pl.BlockSpec((EBS, 16), lambda c, i: (NEB + i, 0)),
            pl.BlockSpec((1, 8, HH), lambda c, i: (c, 0, 0)),
            pl.BlockSpec((1, 1, HH), lambda c, i: (c, 0, 0)),
        ],
        out_specs=pl.BlockSpec((EBS // 128, 128, HH),
                               lambda c, i: (c * NEB + i, 0, 0)),
        out_shape=jax.ShapeDtypeStruct((2 * EB, 128, HH), jnp.float32),
    )(p_all, p_all, wc, b1)


def _p2_body(rlo_ref, rhi_ref, h_ref, dlo_ref, dhi_ref, w2_ref, b2_ref,
             u1a_ref, u1b_ref, bu1_ref, u2_ref, bu2_ref, g_ref, be_ref,
             o_ref):
    R = jnp.concatenate([rlo_ref[...], rhi_ref[...]], axis=1)
    deg = dlo_ref[:, :1] + dhi_ref[:, :1]
    aggr = R @ w2_ref[...] + deg * b2_ref[...]
    h = h_ref[...]
    u_pre = h @ u1a_ref[...] + aggr @ u1b_ref[...] + bu1_ref[...]
    out = jax.nn.relu(u_pre) @ u2_ref[...] + bu2_ref[...]
    mu = jnp.mean(out, axis=-1, keepdims=True)
    var = jnp.mean((out - mu) ** 2, axis=-1, keepdims=True)
    out = (out - mu) / jnp.sqrt(var + 1e-5) * g_ref[...] + be_ref[...]
    o_ref[...] = jax.nn.relu(out) + h


def _p2(r2, h, deg2, w2, b2, u1a, u1b, bu1, u2, bu2, g, be):
    mm = lambda i: (0, 0)
    return pl.pallas_call(
        _p2_body,
        grid=(NB,),
        in_specs=[
            pl.BlockSpec((BS, HH), lambda i: (i, 0)),
            pl.BlockSpec((BS, HH), lambda i: (NB + i, 0)),
            pl.BlockSpec((BS, H), lambda i: (i, 0)),
            pl.BlockSpec((BS, 16), lambda i: (i, 0)),
            pl.BlockSpec((BS, 16), lambda i: (NB + i, 0)),
            pl.BlockSpec((H, H), mm),
            pl.BlockSpec((1, H), mm),
            pl.BlockSpec((H, H), mm),
            pl.BlockSpec((H, H), mm),
            pl.BlockSpec((1, H), mm),
            pl.BlockSpec((H, H), mm),
            pl.BlockSpec((1, H), mm),
            pl.BlockSpec((1, H), mm),
            pl.BlockSpec((1, H), mm),
        ],
        out_specs=pl.BlockSpec((BS, H), lambda i: (i, 0)),
        out_shape=jax.ShapeDtypeStruct((NP, H), jnp.float32),
    )(r2, r2, h, deg2, deg2, w2, b2, u1a, u1b, bu1, u2, bu2, g, be)


def _p3_body(h_ref, bt_ref, wo_ref, bo_ref, g1_ref, bg1_ref, g2_ref,
             bg2_ref, o_ref, acc, cnt):
    i = pl.program_id(0)

    @pl.when(i == 0)
    def _():
        acc[...] = jnp.zeros_like(acc)
        cnt[...] = jnp.zeros_like(cnt)

    xo = h_ref[...] @ wo_ref[...] + bo_ref[...]
    bt = bt_ref[...]
    gid = lax.broadcasted_iota(jnp.int32, (BS, 8), 1)
    rm = (bt == gid).astype(jnp.float32)
    dn = (((0,), (0,)), ((), ()))
    acc[...] += lax.dot_general(rm, xo, dn)
    cnt[...] += lax.dot_general(rm, jnp.ones_like(xo), dn)

    @pl.when(i == NB - 1)
    def _():
        cts = cnt[...]
        means = jnp.where(cts > 0, acc[...] / jnp.maximum(cts, 1.0), 0.0)
        gm = jax.nn.relu(means @ g1_ref[...] + bg1_ref[...])
        o_ref[...] = gm @ g2_ref[...] + bg2_ref[...]


def _p3(h, btp, wo, bo, g1, bg1, g2, bg2):
    mm = lambda i: (0, 0)
    return pl.pallas_call(
        _p3_body,
        grid=(NB,),
        in_specs=[
            pl.BlockSpec((BS, H), lambda i: (i, 0)),
            pl.BlockSpec((BS, 1), lambda i: (i, 0)),
            pl.BlockSpec((H, 128), mm),
            pl.BlockSpec((1, 128), mm),
            pl.BlockSpec((128, H), mm),
            pl.BlockSpec((1, H), mm),
            pl.BlockSpec((H, 128), mm),
            pl.BlockSpec((1, 128), mm),
        ],
        out_specs=pl.BlockSpec((8, 128), mm),
        out_shape=jax.ShapeDtypeStruct((8, 128), jnp.float32),
        scratch_shapes=[
            pltpu.VMEM((8, 128), jnp.float32),
            pltpu.VMEM((8, 128), jnp.float32),
        ],
    )(h, btp, wo, bo, g1, bg1, g2, bg2)


def kernel(x, pos, edge_index, batch, params):
    dsth = edge_index[1].astype(jnp.int32).reshape(EB, 128)
    srch = edge_index[0].astype(jnp.int32).reshape(EB, 128)
    posp = jnp.concatenate([pos, jnp.zeros((N, 13), jnp.float32)], axis=1)
    p_all, deg2 = _sc_posdeg(posp, dsth, srch)

    xp = jnp.concatenate([
        jnp.concatenate([x, jnp.zeros((N, 5), jnp.float32)], axis=1),
        jnp.zeros((NP - N, 8), jnp.float32)], axis=0)
    wi = jnp.concatenate([params["inp"]["w"],
                          jnp.zeros((5, H), jnp.float32)], axis=0)
    h = _p0(xp, wi, params["inp"]["b"][None])

    btp = jnp.concatenate([batch.astype(jnp.int32),
                           jnp.full((NP - N,), 8, jnp.int32)])[:, None]
    zrows = jnp.zeros((NP, HH), jnp.float32)

    for lp in params["layers"]:
        W1 = lp["m1"]["w"]
        wc6 = jnp.concatenate([W1[2 * H:], jnp.zeros((2, H), jnp.float32)],
                              axis=0)
        wc = jnp.stack([wc6[:, :HH], wc6[:, HH:]], axis=0)
        b1 = lp["m1"]["b"][None]
        b1s = jnp.stack([b1[:, :HH], b1[:, HH:]], axis=0)
        wa = jnp.stack([W1[:H, :HH], W1[:H, HH:]], axis=0)
        wb = jnp.stack([W1[H:2 * H, :HH], W1[H:2 * H, HH:]], axis=0)
        a2, b2t = _p1(h, wa, wb)
        cf = _pc(p_all, wc, b1s)
        r2 = _sc_edge_pass(a2, b2t, cf, dsth, srch, zrows)
        h = _p2(r2, h, deg2, lp["m2"]["w"], lp["m2"]["b"][None],
                lp["u1"]["w"][:H], lp["u1"]["w"][H:], lp["u1"]["b"][None],
                lp["u2"]["w"], lp["u2"]["b"][None],
                lp["ln_g"][None], lp["ln_b"][None])

    return _p3(h, btp, params["outp"]["w"], params["outp"]["b"][None],
               params["gp1"]["w"], params["gp1"]["b"][None],
               params["gp2"]["w"], params["gp2"]["b"][None])
